# Initial kernel scaffold; baseline (speedup 1.0000x reference)
import jax
import jax.numpy as jnp
from jax.experimental import pallas as pl


def kernel(in_feat, edge_index, e_features, inlet_src, inlet_dst, in_n_features, in_e_features, outlet_src, outlet_dst, out_n_features, out_e_features, params):
    raise NotImplementedError("write your pallas kernel here")


# speedup vs baseline: 2.8245x; 2.8245x over previous
import functools

import jax
import jax.numpy as jnp
from jax import lax
from jax.experimental import pallas as pl
from jax.experimental.pallas import tpu as pltpu
from jax.experimental.pallas import tpu_sc as plsc

_F32 = jnp.float32
_N_TILES = 32


def _mesh():
    return plsc.VectorSubcoreMesh(
        core_axis_name="c", subcore_axis_name="s", num_cores=2, num_subcores=16
    )


def _dot(a, b, bf16):
    if bf16:
        return jnp.dot(
            a.astype(jnp.bfloat16),
            b.astype(jnp.bfloat16),
            preferred_element_type=_F32,
        )
    return jnp.dot(a, b, preferred_element_type=_F32, precision=lax.Precision.HIGHEST)


def _mlp_tc(groups, wis, bi, wh, bh, wo, bo, g, b, residual, block, bf16=True):
    sizes = [len(gr) for gr in groups]
    xs = [x for gr in groups for x in gr]
    n = xs[0].shape[0]
    nx = len(xs)
    has_ln = g is not None
    has_res = residual is not None
    fo = wo.shape[1]

    def body(*refs):
        xr = refs[:nx]
        wr = refs[nx : nx + len(sizes)]
        k = nx + len(sizes)
        bi_r, wh_r, bh_r, wo_r, bo_r = refs[k : k + 5]
        k += 5
        if has_ln:
            g_r, b_r = refs[k : k + 2]
            k += 2
        if has_res:
            res_r = refs[k]
            k += 1
        out_r = refs[k]

        h = None
        pos = 0
        for gi, sz in enumerate(sizes):
            xg = xr[pos][...]
            for j in range(1, sz):
                xg = xg + xr[pos + j][...]
            pos += sz
            t = _dot(xg, wr[gi][...], bf16)
            h = t if h is None else h + t
        h = h + bi_r[...]
        h = jnp.where(h >= 0, h, 0.01 * h)
        h = _dot(h, wh_r[...], bf16) + bh_r[...]
        h = jnp.where(h >= 0, h, 0.01 * h)
        h = _dot(h, wo_r[...], bf16) + bo_r[...]
        if has_ln:
            mu = jnp.mean(h, axis=-1, keepdims=True)
            d = h - mu
            var = jnp.mean(d * d, axis=-1, keepdims=True)
            h = d * lax.rsqrt(var + 1e-5) * g_r[...] + b_r[...]
        if has_res:
            h = h + res_r[...]
        out_r[...] = h

    grid = n // block
    full = lambda a: pl.BlockSpec(a.shape, lambda i: (0,) * a.ndim)
    row = lambda a: pl.BlockSpec((block, a.shape[1]), lambda i: (i, 0))

    ops = [bi.reshape(1, -1), wh, bh.reshape(1, -1), wo, bo.reshape(1, -1)]
    if has_ln:
        ops += [g.reshape(1, -1), b.reshape(1, -1)]
    args = list(xs) + list(wis) + ops
    specs = [row(x) for x in xs] + [full(w) for w in wis] + [full(o) for o in ops]
    if has_res:
        args.append(residual)
        specs.append(row(residual))

    return pl.pallas_call(
        body,
        grid=(grid,),
        in_specs=specs,
        out_specs=pl.BlockSpec((block, fo), lambda i: (i, 0)),
        out_shape=jax.ShapeDtypeStruct((n, fo), _F32),
    )(*args)


def _mlp_apply(p, xs, residual, block, bf16=True):
    wis = []
    off = 0
    for x in xs:
        wis.append(p["Wi"][off : off + x.shape[1]])
        off += x.shape[1]
    g = p.get("g")
    b = p.get("b")
    return _mlp_tc(
        [[x] for x in xs], wis, p["bi"], p["Wh"][0], p["bh"][0], p["Wo"], p["bo"],
        g, b, residual, block, bf16=bf16,
    )


def _bc_encode_tc(e_feat, n_feat, src_idx, p, side):
    ne = e_feat.shape[0]
    wi_e = p["Wi"][: e_feat.shape[1]]
    wi_n = p["Wi"][e_feat.shape[1] :]

    def body(e_r, n_r, s_r, we_r, wn_r, bi_r, wh_r, bh_r, wo_r, bo_r, g_r, b_r, out_r):
        sel = lax.broadcasted_iota(jnp.int32, (ne, n_feat.shape[0]), 1)
        onehot = jnp.where(s_r[...] == sel, 1.0, 0.0).astype(_F32)
        xn = jnp.dot(onehot, n_r[...], preferred_element_type=_F32, precision=lax.Precision.HIGHEST)
        h = jnp.dot(e_r[...], we_r[...], preferred_element_type=_F32, precision=lax.Precision.HIGHEST)
        h = h + jnp.dot(xn, wn_r[...], preferred_element_type=_F32, precision=lax.Precision.HIGHEST) + bi_r[...]
        h = jnp.where(h >= 0, h, 0.01 * h)
        h = jnp.dot(h, wh_r[...], preferred_element_type=_F32, precision=lax.Precision.HIGHEST) + bh_r[...]
        h = jnp.where(h >= 0, h, 0.01 * h)
        h = jnp.dot(h, wo_r[...], preferred_element_type=_F32, precision=lax.Precision.HIGHEST) + bo_r[...]
        mu = jnp.mean(h, axis=-1, keepdims=True)
        d = h - mu
        var = jnp.mean(d * d, axis=-1, keepdims=True)
        h = d * lax.rsqrt(var + 1e-5) * g_r[...] + b_r[...]
        z = jnp.zeros_like(h)
        out_r[...] = jnp.concatenate([h, z] if side == 0 else [z, h], axis=1)

    args = [
        e_feat,
        n_feat,
        src_idx.reshape(ne, 1),
        wi_e,
        wi_n,
        p["bi"].reshape(1, -1),
        p["Wh"][0],
        p["bh"][0].reshape(1, -1),
        p["Wo"],
        p["bo"].reshape(1, -1),
        p["g"].reshape(1, -1),
        p["b"].reshape(1, -1),
    ]
    return pl.pallas_call(
        body,
        out_shape=jax.ShapeDtypeStruct((ne, 16), _F32),
    )(*args)


def _sc_gather(table, src, dst, chunk):
    ne = src.shape[0]
    d = table.shape[1]
    per_tile = ne // _N_TILES
    n_chunks = per_tile // chunk

    @functools.partial(
        pl.kernel,
        out_type=[
            jax.ShapeDtypeStruct((ne, d), _F32),
            jax.ShapeDtypeStruct((ne, d), _F32),
        ],
        mesh=_mesh(),
        compiler_params=pltpu.CompilerParams(use_tc_tiling_on_sc=False),
        scratch_types=[
            pltpu.VMEM((chunk,), jnp.int32),
            pltpu.VMEM((chunk, d), _F32),
            pltpu.VMEM((chunk,), jnp.int32),
            pltpu.VMEM((chunk, d), _F32),
            pltpu.SemaphoreType.DMA,
            pltpu.SemaphoreType.DMA,
        ],
    )
    def k(table_h, src_h, dst_h, gsrc_h, gdst_h, si, sv, di, dv, sem_a, sem_b):
        wid = lax.axis_index("c") * 16 + lax.axis_index("s")
        base = wid * per_tile

        def step(j, carry):
            off = base + j * chunk
            pltpu.sync_copy(src_h.at[pl.ds(off, chunk)], si)
            pltpu.sync_copy(dst_h.at[pl.ds(off, chunk)], di)
            cp_a = pltpu.async_copy(table_h.at[si], sv, sem_a)
            cp_b = pltpu.async_copy(table_h.at[di], dv, sem_b)
            cp_a.wait()
            cp_b.wait()
            pltpu.sync_copy(sv, gsrc_h.at[pl.ds(off, chunk), :])
            pltpu.sync_copy(dv, gdst_h.at[pl.ds(off, chunk), :])
            return carry

        lax.fori_loop(0, n_chunks, step, 0)

    return k(table, src, dst)


def _sc_segment_sum(rows, idx, n_seg, chunk):
    ne = rows.shape[0]
    d = rows.shape[1]
    per_tile = ne // _N_TILES
    n_chunks = per_tile // chunk
    slab = n_seg // 16

    @functools.partial(
        pl.kernel,
        out_type=jax.ShapeDtypeStruct((2 * n_seg, d), _F32),
        mesh=_mesh(),
        compiler_params=pltpu.CompilerParams(use_tc_tiling_on_sc=False),
        scratch_types=[
            pltpu.VMEM((chunk,), jnp.int32),
            pltpu.VMEM((chunk, d), _F32),
            pltpu.VMEM_SHARED((n_seg, d), _F32),
        ],
    )
    def k(rows_h, idx_h, zeros_h, out_h, ib, rb, accum):
        c = lax.axis_index("c")
        s = lax.axis_index("s")
        wid = c * 16 + s
        base = wid * per_tile

        pltpu.sync_copy(zeros_h, accum.at[pl.ds(s * slab, slab), :])
        plsc.subcore_barrier()

        def step(j, carry):
            off = base + j * chunk
            pltpu.sync_copy(idx_h.at[pl.ds(off, chunk)], ib)
            pltpu.sync_copy(rows_h.at[pl.ds(off, chunk), :], rb)
            pltpu.sync_copy(rb, accum.at[ib], add=True)
            return carry

        lax.fori_loop(0, n_chunks, step, 0)
        plsc.subcore_barrier()
        pltpu.sync_copy(
            accum.at[pl.ds(s * slab, slab), :],
            out_h.at[pl.ds(c * n_seg + s * slab, slab), :],
        )

    zeros = jnp.zeros((slab, d), _F32)
    return k(rows, idx, zeros)


def kernel(in_feat, edge_index, e_features, inlet_src, inlet_dst, in_n_features, in_e_features, outlet_src, outlet_dst, out_n_features, out_e_features, params):
    n_nodes = in_feat.shape[0]
    src = edge_index[0]
    dst = edge_index[1]

    proc_node = _mlp_apply(params["enc_nodes"], [in_feat], None, block=5000, bf16=False)
    proc_edge = _mlp_apply(params["enc_edges"], [e_features], None, block=8000)

    bc_in = _bc_encode_tc(in_e_features, in_n_features, inlet_src, params["enc_inlet"], 0)
    bc_out = _bc_encode_tc(out_e_features, out_n_features, outlet_src, params["enc_outlet"], 1)
    bc_rows = jnp.concatenate([bc_in, bc_out], axis=0)
    bc_dst = jnp.concatenate([inlet_dst, outlet_dst], axis=0)
    bc = _sc_segment_sum(bc_rows, bc_dst, n_nodes, chunk=64)
    bc0 = bc[:n_nodes]
    bc1 = bc[n_nodes:]

    for i in range(3):
        gsrc, gdst = _sc_gather(proc_node, src, dst, chunk=1000)
        proc_edge = _mlp_apply(
            params["proc_e"][i], [proc_edge, gsrc, gdst], proc_edge, block=8000
        )
        ps = _sc_segment_sum(proc_edge, dst, n_nodes, chunk=1000)
        ps0 = ps[:n_nodes]
        ps1 = ps[n_nodes:]
        p = params["proc_n"][i]
        wi = p["Wi"]
        wis = [wi[0:16], wi[16:32], wi[32:48]]
        proc_node = _mlp_tc(
            [[proc_node], [ps0, ps1], [bc0, bc1]],
            wis,
            p["bi"],
            p["Wh"][0],
            p["bh"][0],
            p["Wo"],
            p["bo"],
            p["g"],
            p["b"],
            proc_node,
            block=5000,
            bf16=True,
        )

    return _mlp_apply(params["output"], [proc_node], None, block=5000, bf16=True)
